# baseline (speedup 1.0000x reference)
import functools

import jax
import jax.numpy as jnp
from jax import lax
from jax.experimental import pallas as pl
from jax.experimental.pallas import tpu as pltpu
from jax.experimental.pallas import tpu_sc as plsc

VOCAB = 100000
HIDDEN = 128
MAX_POS = 512
EPS = 1e-12

NC, NS, L = 2, 16, 16
NW = NC * NS
N_TOK = 64 * 512
C = 128

NSPLIT = 4
N_PART = N_TOK // NSPLIT

BT = 8192
NSEQ_BLK = BT // MAX_POS


def _make_gather_body(n_tok):
    tpw = n_tok // NW
    nchunk = tpw // C

    def body(ids_hbm, word_hbm, out_hbm, rows0, rows1, idx_all,
             gsem0, gsem1, wsem0, wsem1):
        wid = lax.axis_index("s") * NC + lax.axis_index("c")
        base = wid * tpw
        pltpu.sync_copy(ids_hbm.at[pl.ds(base, tpw)], idx_all)

        def prep(c, rowsv, gsem):
            pltpu.async_copy(
                word_hbm.at[idx_all.at[pl.ds(c * C, C)]], rowsv, gsem)

        def gwait(c, rowsv, gsem):
            pltpu.make_async_copy(
                word_hbm.at[idx_all.at[pl.ds(c * C, C)]], rowsv, gsem).wait()

        def wb_start(c, rowsv, wsem):
            start = base + c * C
            pltpu.async_copy(rowsv, out_hbm.at[pl.ds(start, C)], wsem)

        def wb_wait(rowsv, wsem):
            pltpu.make_async_copy(
                rowsv, out_hbm.at[pl.ds(base, C)], wsem).wait()

        prep(0, rows0, gsem0)

        def pair(h, carry):
            c0 = 2 * h

            @pl.when(h > 0)
            def _():
                wb_wait(rows1, wsem1)

            prep(c0 + 1, rows1, gsem1)
            gwait(c0, rows0, gsem0)
            wb_start(c0, rows0, wsem0)

            @pl.when(h < nchunk // 2 - 1)
            def _():
                wb_wait(rows0, wsem0)
                prep(c0 + 2, rows0, gsem0)

            gwait(c0 + 1, rows1, gsem1)
            wb_start(c0 + 1, rows1, wsem1)
            return carry

        lax.fori_loop(0, nchunk // 2, pair, 0)
        wb_wait(rows0, wsem0)
        wb_wait(rows1, wsem1)

    return body


def _sc_gather(ids_flat, word_table, n_tok):
    mesh = plsc.VectorSubcoreMesh(core_axis_name="c", subcore_axis_name="s")
    run = functools.partial(
        pl.kernel,
        out_type=jax.ShapeDtypeStruct((n_tok, HIDDEN), jnp.float32),
        mesh=mesh,
        compiler_params=pltpu.CompilerParams(needs_layout_passes=False),
        scratch_types=[
            pltpu.VMEM((C, HIDDEN), jnp.float32),
            pltpu.VMEM((C, HIDDEN), jnp.float32),
            pltpu.VMEM((n_tok // NW,), jnp.int32),
            pltpu.SemaphoreType.DMA,
            pltpu.SemaphoreType.DMA,
            pltpu.SemaphoreType.DMA,
            pltpu.SemaphoreType.DMA,
        ],
    )(_make_gather_body(n_tok))
    return run(ids_flat, word_table)


def _ln_math(x_ref, ttf_ref, pos_ref, type_ref, g_ref, b_ref, o_ref, obase):
    pos = pos_ref[...]
    t0 = type_ref[0, :][None, :]
    td = type_ref[1, :][None, :] - t0
    gv = g_ref[...]
    bv = b_ref[...]
    ones = jnp.full((HIDDEN, HIDDEN), 1.0 / HIDDEN, dtype=jnp.float32)
    for s in range(NSEQ_BLK):
        sl = pl.ds(MAX_POS * s, MAX_POS)
        osl = pl.ds(obase + MAX_POS * s, MAX_POS)
        ttf = ttf_ref[0, :, s][:, None]
        x = x_ref[sl, :] + pos + t0 + ttf * td
        mu = jnp.dot(x, ones, preferred_element_type=jnp.float32)
        ex2 = jnp.dot(x * x, ones, preferred_element_type=jnp.float32)
        var = ex2 - mu * mu
        o_ref[osl, :] = (x - mu) * lax.rsqrt(var + EPS) * gv + bv


def _ln_body0(x_ref, ttf_ref, pos_ref, type_ref, g_ref, b_ref, o_ref):
    _ln_math(x_ref, ttf_ref, pos_ref, type_ref, g_ref, b_ref, o_ref, 0)


def _ln_body1(prev_ref, x_ref, ttf_ref, pos_ref, type_ref, g_ref, b_ref,
              o_ref):
    del prev_ref
    _ln_math(x_ref, ttf_ref, pos_ref, type_ref, g_ref, b_ref, o_ref, 0)


_LN_TAIL_SPECS = [
    pl.BlockSpec((1, MAX_POS, NSEQ_BLK), lambda i: (i, 0, 0)),
    pl.BlockSpec((MAX_POS, HIDDEN), lambda i: (0, 0)),
    pl.BlockSpec((2, HIDDEN), lambda i: (0, 0)),
    pl.BlockSpec((1, HIDDEN), lambda i: (0, 0)),
    pl.BlockSpec((1, HIDDEN), lambda i: (0, 0)),
]


@jax.jit
def _bert_embed(ids_flat, ttf3, word_table, pos_table, type_table, gamma,
                beta):
    g2 = gamma.reshape(1, HIDDEN)
    b2 = beta.reshape(1, HIDDEN)
    nblk_part = N_PART // BT
    xgs = [_sc_gather(ids_flat[i * N_PART:(i + 1) * N_PART], word_table,
                      N_PART)
           for i in range(NSPLIT)]
    out = None
    for p in range(NSPLIT):
        x_spec = pl.BlockSpec((BT, HIDDEN), lambda i: (i, 0))
        o_spec = pl.BlockSpec(
            (BT, HIDDEN), lambda i, _p=p: (i + _p * nblk_part, 0))
        tt_part = ttf3[p * nblk_part:(p + 1) * nblk_part]
        common = dict(
            grid=(nblk_part,),
            out_specs=o_spec,
            out_shape=jax.ShapeDtypeStruct((N_TOK, HIDDEN), jnp.float32),
            compiler_params=pltpu.CompilerParams(
                dimension_semantics=("arbitrary",)),
        )
        if p == 0:
            out = pl.pallas_call(
                _ln_body0,
                in_specs=[x_spec] + _LN_TAIL_SPECS,
                **common,
            )(xgs[0], tt_part, pos_table, type_table, g2, b2)
        else:
            out = pl.pallas_call(
                _ln_body1,
                in_specs=[pl.BlockSpec(memory_space=pltpu.MemorySpace.HBM),
                          x_spec] + _LN_TAIL_SPECS,
                input_output_aliases={0: 0},
                **common,
            )(out, xgs[p], tt_part, pos_table, type_table, g2, b2)
    return out


def kernel(input_ids, token_type_ids, word_table, pos_table, type_table,
           gamma, beta):
    B, S = input_ids.shape
    out = _bert_embed(
        input_ids.reshape(-1).astype(jnp.int32),
        token_type_ids.astype(jnp.float32).reshape(
            N_TOK // BT, NSEQ_BLK, S).transpose(0, 2, 1),
        word_table, pos_table, type_table, gamma, beta)
    return out.reshape(B, S, HIDDEN)


# speedup vs baseline: 1.1325x; 1.1325x over previous
import functools

import jax
import jax.numpy as jnp
from jax import lax
from jax.experimental import pallas as pl
from jax.experimental.pallas import tpu as pltpu
from jax.experimental.pallas import tpu_sc as plsc

VOCAB = 100000
HIDDEN = 128
MAX_POS = 512
EPS = 1e-12

NC, NS, L = 2, 16, 16
NW = NC * NS
N_TOK = 64 * 512
C = 128

NSPLIT = 2
N_PART = N_TOK // NSPLIT

BT = 8192
NSEQ_BLK = BT // MAX_POS


def _make_gather_body(n_tok):
    tpw = n_tok // NW
    nchunk = tpw // C

    def body(ids_hbm, word_hbm, out_hbm, rows0, rows1, idx_all,
             gsem0, gsem1, wsem0, wsem1):
        wid = lax.axis_index("s") * NC + lax.axis_index("c")
        base = wid * tpw
        pltpu.sync_copy(ids_hbm.at[pl.ds(base, tpw)], idx_all)

        def prep(c, rowsv, gsem):
            pltpu.async_copy(
                word_hbm.at[idx_all.at[pl.ds(c * C, C)]], rowsv, gsem)

        def gwait(c, rowsv, gsem):
            pltpu.make_async_copy(
                word_hbm.at[idx_all.at[pl.ds(c * C, C)]], rowsv, gsem).wait()

        def wb_start(c, rowsv, wsem):
            start = base + c * C
            pltpu.async_copy(rowsv, out_hbm.at[pl.ds(start, C)], wsem)

        def wb_wait(rowsv, wsem):
            pltpu.make_async_copy(
                rowsv, out_hbm.at[pl.ds(base, C)], wsem).wait()

        prep(0, rows0, gsem0)

        def pair(h, carry):
            c0 = 2 * h

            @pl.when(h > 0)
            def _():
                wb_wait(rows1, wsem1)

            prep(c0 + 1, rows1, gsem1)
            gwait(c0, rows0, gsem0)
            wb_start(c0, rows0, wsem0)

            @pl.when(h < nchunk // 2 - 1)
            def _():
                wb_wait(rows0, wsem0)
                prep(c0 + 2, rows0, gsem0)

            gwait(c0 + 1, rows1, gsem1)
            wb_start(c0 + 1, rows1, wsem1)
            return carry

        lax.fori_loop(0, nchunk // 2, pair, 0)
        wb_wait(rows0, wsem0)
        wb_wait(rows1, wsem1)

    return body


def _sc_gather(ids_flat, word_table, n_tok):
    mesh = plsc.VectorSubcoreMesh(core_axis_name="c", subcore_axis_name="s")
    run = functools.partial(
        pl.kernel,
        out_type=jax.ShapeDtypeStruct((n_tok, HIDDEN), jnp.float32),
        mesh=mesh,
        compiler_params=pltpu.CompilerParams(needs_layout_passes=False),
        scratch_types=[
            pltpu.VMEM((C, HIDDEN), jnp.float32),
            pltpu.VMEM((C, HIDDEN), jnp.float32),
            pltpu.VMEM((n_tok // NW,), jnp.int32),
            pltpu.SemaphoreType.DMA,
            pltpu.SemaphoreType.DMA,
            pltpu.SemaphoreType.DMA,
            pltpu.SemaphoreType.DMA,
        ],
    )(_make_gather_body(n_tok))
    return run(ids_flat, word_table)


def _ln_math(x_ref, ttf_ref, pos_ref, type_ref, g_ref, b_ref, o_ref, obase):
    pos = pos_ref[...]
    t0 = type_ref[0, :][None, :]
    td = type_ref[1, :][None, :] - t0
    gv = g_ref[...]
    bv = b_ref[...]
    ones = jnp.full((HIDDEN, HIDDEN), 1.0 / HIDDEN, dtype=jnp.float32)
    for s in range(NSEQ_BLK):
        sl = pl.ds(MAX_POS * s, MAX_POS)
        osl = pl.ds(obase + MAX_POS * s, MAX_POS)
        ttf = ttf_ref[0, :, s][:, None]
        x = x_ref[sl, :] + pos + t0 + ttf * td
        mu = jnp.dot(x, ones, preferred_element_type=jnp.float32)
        ex2 = jnp.dot(x * x, ones, preferred_element_type=jnp.float32)
        var = ex2 - mu * mu
        o_ref[osl, :] = (x - mu) * lax.rsqrt(var + EPS) * gv + bv


def _ln_body0(x_ref, ttf_ref, pos_ref, type_ref, g_ref, b_ref, o_ref):
    _ln_math(x_ref, ttf_ref, pos_ref, type_ref, g_ref, b_ref, o_ref, 0)


def _ln_body1(prev_ref, x_ref, ttf_ref, pos_ref, type_ref, g_ref, b_ref,
              o_ref):
    del prev_ref
    _ln_math(x_ref, ttf_ref, pos_ref, type_ref, g_ref, b_ref, o_ref, 0)


_LN_TAIL_SPECS = [
    pl.BlockSpec((1, MAX_POS, NSEQ_BLK), lambda i: (i, 0, 0)),
    pl.BlockSpec((MAX_POS, HIDDEN), lambda i: (0, 0)),
    pl.BlockSpec((2, HIDDEN), lambda i: (0, 0)),
    pl.BlockSpec((1, HIDDEN), lambda i: (0, 0)),
    pl.BlockSpec((1, HIDDEN), lambda i: (0, 0)),
]


@jax.jit
def _bert_embed(ids_flat, ttf3, word_table, pos_table, type_table, gamma,
                beta):
    g2 = gamma.reshape(1, HIDDEN)
    b2 = beta.reshape(1, HIDDEN)
    nblk_part = N_PART // BT
    xgs = [_sc_gather(ids_flat[i * N_PART:(i + 1) * N_PART], word_table,
                      N_PART)
           for i in range(NSPLIT)]
    out = None
    for p in range(NSPLIT):
        x_spec = pl.BlockSpec((BT, HIDDEN), lambda i: (i, 0))
        o_spec = pl.BlockSpec(
            (BT, HIDDEN), lambda i, _p=p: (i + _p * nblk_part, 0))
        tt_part = ttf3[p * nblk_part:(p + 1) * nblk_part]
        common = dict(
            grid=(nblk_part,),
            out_specs=o_spec,
            out_shape=jax.ShapeDtypeStruct((N_TOK, HIDDEN), jnp.float32),
            compiler_params=pltpu.CompilerParams(
                dimension_semantics=("arbitrary",)),
        )
        if p == 0:
            out = pl.pallas_call(
                _ln_body0,
                in_specs=[x_spec] + _LN_TAIL_SPECS,
                **common,
            )(xgs[0], tt_part, pos_table, type_table, g2, b2)
        else:
            out = pl.pallas_call(
                _ln_body1,
                in_specs=[pl.BlockSpec(memory_space=pltpu.MemorySpace.HBM),
                          x_spec] + _LN_TAIL_SPECS,
                input_output_aliases={0: 0},
                **common,
            )(out, xgs[p], tt_part, pos_table, type_table, g2, b2)
    return out


def kernel(input_ids, token_type_ids, word_table, pos_table, type_table,
           gamma, beta):
    B, S = input_ids.shape
    out = _bert_embed(
        input_ids.reshape(-1).astype(jnp.int32),
        token_type_ids.astype(jnp.float32).reshape(
            N_TOK // BT, NSEQ_BLK, S).transpose(0, 2, 1),
        word_table, pos_table, type_table, gamma, beta)
    return out.reshape(B, S, HIDDEN)
